# baseline (speedup 1.0000x reference)
import jax
import jax.numpy as jnp
from jax.experimental import pallas as pl
from jax.experimental.pallas import tpu as pltpu

BM = 1024
BN = 2048
BK = 2048


def _mm_kernel(x_ref, w_ref, b_ref, o_ref):
    k = pl.program_id(2)
    acc = jax.lax.dot_general(
        x_ref[...],
        w_ref[...].astype(jnp.float32),
        dimension_numbers=(((1,), (1,)), ((), ())),
        preferred_element_type=jnp.float32,
    )

    o_ref[...] = acc + jnp.where(k == 0, b_ref[...], o_ref[...])


def kernel(x, W, b):
    M, K = x.shape
    N = W.shape[0]
    Wb = W.astype(jnp.bfloat16)
    b2 = b.reshape(1, N)
    nj = N // BN
    nk = K // BK
    grid = (M // BM, nj, nk)

    def _snake(i, j, k):
        j_eff = jnp.where(i % 2 == 1, nj - 1 - j, j)
        k_eff = jnp.where(j % 2 == 1, nk - 1 - k, k)
        return j_eff, k_eff

    return pl.pallas_call(
        _mm_kernel,
        grid=grid,
        in_specs=[
            pl.BlockSpec((BM, BK), lambda i, j, k: (i, _snake(i, j, k)[1])),
            pl.BlockSpec((BN, BK), lambda i, j, k: _snake(i, j, k)),
            pl.BlockSpec((1, BN), lambda i, j, k: (0, _snake(i, j, k)[0])),
        ],
        out_specs=pl.BlockSpec((BM, BN), lambda i, j, k: (i, _snake(i, j, k)[0])),
        out_shape=jax.ShapeDtypeStruct((M, N), jnp.float32),
        compiler_params=pltpu.CompilerParams(
            dimension_semantics=("parallel", "parallel", "arbitrary"),
        ),
    )(x, Wb, b2)
